# baseline (speedup 1.0000x reference)
import functools

import jax
import jax.numpy as jnp
from jax import lax
from jax.experimental import pallas as pl
from jax.experimental.pallas import tpu as pltpu
from jax.experimental.pallas import tpu_sc as plsc

_BS = 256
_NW = 32


def _sc_dispatch(p0, p1, x_i32, T, R, Dw):
    toks_per_w = T // _NW
    CH = 64
    nch = toks_per_w // CH
    mesh = plsc.VectorSubcoreMesh(core_axis_name="c", subcore_axis_name="s")

    @functools.partial(
        pl.kernel,
        mesh=mesh,
        out_type=jax.ShapeDtypeStruct((R, Dw), jnp.int32),
        scratch_types=[
            pltpu.VMEM((CH,), jnp.int32),
            pltpu.VMEM((CH,), jnp.int32),
            pltpu.VMEM((CH, Dw), jnp.int32),
            pltpu.SemaphoreType.DMA,
        ],
    )
    def k(p0_hbm, p1_hbm, x_hbm, out_hbm, i0_v, i1_v, rows_v, sem):
        wid = lax.axis_index("s") * 2 + lax.axis_index("c")
        base = wid * toks_per_w
        for c in range(nch):
            off = base + c * CH
            pltpu.sync_copy(x_hbm.at[pl.ds(off, CH)], rows_v)
            pltpu.sync_copy(p0_hbm.at[pl.ds(off, CH)], i0_v)
            pltpu.sync_copy(p1_hbm.at[pl.ds(off, CH)], i1_v)
            c0 = pltpu.async_copy(rows_v, out_hbm.at[i0_v], sem)
            c1 = pltpu.async_copy(rows_v, out_hbm.at[i1_v], sem)
            c0.wait()
            c1.wait()

    return k(p0, p1, x_i32)


def _sc_combine(p0, p1, w0b, w1b, ys, T, D):
    toks_per_w = T // _NW
    CH = 32
    nch = toks_per_w // CH
    ncol = D // 16
    mesh = plsc.VectorSubcoreMesh(core_axis_name="c", subcore_axis_name="s")

    @functools.partial(
        pl.kernel,
        mesh=mesh,
        out_type=jax.ShapeDtypeStruct((T, D), jnp.float32),
        scratch_types=[
            pltpu.VMEM((CH,), jnp.int32),
            pltpu.VMEM((CH,), jnp.int32),
            pltpu.VMEM((CH, 16), jnp.float32),
            pltpu.VMEM((CH, 16), jnp.float32),
            pltpu.VMEM((CH, D), jnp.float32),
            pltpu.VMEM((CH, D), jnp.float32),
            pltpu.SemaphoreType.DMA,
        ],
    )
    def k(p0_hbm, p1_hbm, w0_hbm, w1_hbm, ys_hbm, out_hbm,
          i0_v, i1_v, w0_v, w1_v, r0_v, r1_v, sem):
        wid = lax.axis_index("s") * 2 + lax.axis_index("c")
        base = wid * toks_per_w
        for c in range(nch):
            off = base + c * CH
            pltpu.sync_copy(p0_hbm.at[pl.ds(off, CH)], i0_v)
            pltpu.sync_copy(p1_hbm.at[pl.ds(off, CH)], i1_v)
            pltpu.sync_copy(w0_hbm.at[pl.ds(off, CH)], w0_v)
            pltpu.sync_copy(w1_hbm.at[pl.ds(off, CH)], w1_v)
            g0 = pltpu.async_copy(ys_hbm.at[i0_v], r0_v, sem)
            g1 = pltpu.async_copy(ys_hbm.at[i1_v], r1_v, sem)
            g0.wait()
            g1.wait()

            def row_body(r, _):
                wr0 = w0_v[r, :]
                wr1 = w1_v[r, :]

                def col_body(j, _):
                    sl = pl.ds(j * 16, 16)
                    r0_v[r, sl] = r0_v[r, sl] * wr0 + r1_v[r, sl] * wr1
                    return 0

                return lax.fori_loop(0, ncol, col_body, 0)

            lax.fori_loop(0, CH, row_body, 0)
            pltpu.sync_copy(r0_v, out_hbm.at[pl.ds(off, CH)])

    return k(p0, p1, w0b, w1b, ys)


def _gmm_body(be_ref, xs_ref, w1_ref, b1_ref, w2_ref, b2_ref, out_ref):
    h = jnp.dot(xs_ref[...], w1_ref[0], preferred_element_type=jnp.float32)
    h = jnp.maximum(h + b1_ref[0], 0.0).astype(jnp.bfloat16)
    y = jnp.dot(h, w2_ref[0], preferred_element_type=jnp.float32)
    out_ref[...] = y + b2_ref[0]


def _gmm(block_expert, xs, W1, b1, W2, b2, nblk, R, D, H):
    grid_spec = pltpu.PrefetchScalarGridSpec(
        num_scalar_prefetch=1,
        grid=(nblk,),
        in_specs=[
            pl.BlockSpec((_BS, D), lambda i, be: (i, 0)),
            pl.BlockSpec((1, D, H), lambda i, be: (be[i], 0, 0)),
            pl.BlockSpec((1, 1, H), lambda i, be: (be[i], 0, 0)),
            pl.BlockSpec((1, H, D), lambda i, be: (be[i], 0, 0)),
            pl.BlockSpec((1, 1, D), lambda i, be: (be[i], 0, 0)),
        ],
        out_specs=pl.BlockSpec((_BS, D), lambda i, be: (i, 0)),
    )
    return pl.pallas_call(
        _gmm_body,
        grid_spec=grid_spec,
        out_shape=jax.ShapeDtypeStruct((R, D), jnp.float32),
        compiler_params=pltpu.CompilerParams(
            dimension_semantics=("arbitrary",),
            vmem_limit_bytes=100 * 1024 * 1024,
        ),
    )(block_expert, xs, W1, b1, W2, b2)


def kernel(x, Wg, bg, W1, b1, W2, b2):
    B, S, D = x.shape
    E = Wg.shape[1]
    H = W1.shape[2]
    K = 2
    T = B * S
    nblk = (K * T) // _BS + E
    R = nblk * _BS

    gate_scores = jnp.einsum('bsd,de->bse', x, Wg) + bg
    ar = jnp.arange(E, dtype=jnp.int32)
    i0 = jnp.argmax(gate_scores, axis=-1)
    v0 = jnp.max(gate_scores, axis=-1)
    masked = jnp.where(ar[None, None, :] == i0[..., None], -jnp.inf, gate_scores)
    i1 = jnp.argmax(masked, axis=-1)
    v1 = jnp.max(masked, axis=-1)
    topk_w = jax.nn.softmax(jnp.stack([v0, v1], axis=-1), axis=-1)

    e_flat = jnp.stack([i0.reshape(T), i1.reshape(T)], axis=-1) \
                .reshape(T * K).astype(jnp.int32)
    onehot = (e_flat[:, None] == ar[None, :]).astype(jnp.int32)
    csum = jnp.cumsum(onehot, axis=0)
    counts = csum[-1]
    rank = jnp.sum(csum * onehot, axis=1) - 1
    blocks_per_e = (counts + _BS - 1) // _BS
    off_blocks = jnp.cumsum(blocks_per_e).astype(jnp.int32)
    row_off = jnp.sum(
        onehot * jnp.concatenate([jnp.zeros((1,), jnp.int32),
                                  off_blocks[:-1]])[None, :], axis=1) * _BS
    pos = row_off + rank
    pos2 = pos.reshape(T, K)
    p0 = pos2[:, 0]
    p1 = pos2[:, 1]
    block_expert = jnp.minimum(
        jnp.sum((off_blocks[None, :] <=
                 jnp.arange(nblk, dtype=jnp.int32)[:, None]).astype(jnp.int32),
                axis=1),
        E - 1).astype(jnp.int32)

    x_i32 = lax.bitcast_convert_type(
        x.reshape(T, D).astype(jnp.bfloat16).reshape(T, D // 2, 2), jnp.int32)
    xs_i32 = _sc_dispatch(p0, p1, x_i32, T, R, D // 2)
    xs = lax.bitcast_convert_type(xs_i32, jnp.bfloat16).reshape(R, D)

    ys = _gmm(block_expert, xs,
              W1.astype(jnp.bfloat16), b1.reshape(E, 1, H),
              W2.astype(jnp.bfloat16), b2.reshape(E, 1, D),
              nblk, R, D, H)

    w0b = jnp.broadcast_to(topk_w[..., 0].reshape(T)[:, None], (T, 16))
    w1b = jnp.broadcast_to(topk_w[..., 1].reshape(T)[:, None], (T, 16))
    out = _sc_combine(p0, p1, w0b, w1b, ys, T, D)
    return out.reshape(B, S, D)
